# speedup vs baseline: 1.0229x; 1.0229x over previous
import functools

import jax
import jax.numpy as jnp
from jax.experimental import pallas as pl
from jax.experimental.pallas import tpu as pltpu

_T = 2048
_H = 2048
_DFF = 768
_E = 8
_EPAD = 128
_BT = 256


def _router_body(x_ref, gwt_ref, w_ref):
    x = x_ref[...]
    logits = jax.lax.dot_general(
        x, gwt_ref[...], (((1,), (0,)), ((), ())),
        preferred_element_type=jnp.float32)
    col = jax.lax.broadcasted_iota(jnp.int32, (_BT, _EPAD), 1)
    valid = col < _E
    neg = jnp.float32(-1e30)
    logits = jnp.where(valid, logits, neg)
    m = jnp.max(logits, axis=1, keepdims=True)
    p = jnp.exp(logits - m)
    p = p / jnp.sum(p, axis=1, keepdims=True)
    p = jnp.where(valid, p, -1.0)
    m1 = jnp.max(p, axis=1, keepdims=True)
    i1 = jnp.min(jnp.where(p == m1, col, _EPAD), axis=1, keepdims=True)
    sel1 = col == i1
    p2 = jnp.where(sel1, -1.0, p)
    m2 = jnp.max(p2, axis=1, keepdims=True)
    i2 = jnp.min(jnp.where(p2 == m2, col, _EPAD), axis=1, keepdims=True)
    sel2 = col == i2
    denom = m1 + m2
    w1 = m1 / denom
    w2 = m2 / denom
    w_ref[...] = jnp.where(sel1, w1, 0.0) + jnp.where(sel2, w2, 0.0)


def _router(x, gate_w):
    gwt = jnp.zeros((_H, _EPAD), jnp.float32).at[:, :_E].set(gate_w.T)
    return pl.pallas_call(
        _router_body,
        grid=(_T // _BT,),
        in_specs=[
            pl.BlockSpec((_BT, _H), lambda i: (i, 0)),
            pl.BlockSpec((_H, _EPAD), lambda i: (0, 0)),
        ],
        out_specs=pl.BlockSpec((_BT, _EPAD), lambda i: (i, 0)),
        out_shape=jax.ShapeDtypeStruct((_T, _EPAD), jnp.float32),
    )(x, gwt)


def _ffn_body(x_ref, w_ref, gp_ref, up_ref, dp_ref, out_ref):
    e = pl.program_id(1)
    xb = x_ref[...]
    g = jax.lax.dot_general(
        xb, gp_ref[0], (((1,), (0,)), ((), ())),
        preferred_element_type=jnp.float32)
    u = jax.lax.dot_general(
        xb, up_ref[0], (((1,), (0,)), ((), ())),
        preferred_element_type=jnp.float32)
    h = (g * jax.lax.logistic(g) * u).astype(jnp.bfloat16)
    y = jax.lax.dot_general(
        h, dp_ref[0], (((1,), (0,)), ((), ())),
        preferred_element_type=jnp.float32)
    col = jax.lax.broadcasted_iota(jnp.int32, (_BT, _EPAD), 1)
    we = jnp.sum(jnp.where(col == e, w_ref[...], 0.0), axis=1, keepdims=True)
    y = y * we

    @pl.when(e == 0)
    def _():
        out_ref[...] = y

    @pl.when(e != 0)
    def _():
        out_ref[...] += y


def _ffn(xb, w, gpt, upt, dpt):
    return pl.pallas_call(
        _ffn_body,
        grid=(_T // _BT, _E),
        in_specs=[
            pl.BlockSpec((_BT, _H), lambda i, e: (i, 0)),
            pl.BlockSpec((_BT, _EPAD), lambda i, e: (i, 0)),
            pl.BlockSpec((1, _H, _DFF), lambda i, e: (e, 0, 0)),
            pl.BlockSpec((1, _H, _DFF), lambda i, e: (e, 0, 0)),
            pl.BlockSpec((1, _DFF, _H), lambda i, e: (e, 0, 0)),
        ],
        out_specs=pl.BlockSpec((_BT, _H), lambda i, e: (i, 0)),
        out_shape=jax.ShapeDtypeStruct((_T, _H), jnp.float32),
        compiler_params=pltpu.CompilerParams(
            dimension_semantics=("arbitrary", "arbitrary")),
    )(xb, w, gpt, upt, dpt)


@jax.jit
def kernel(hidden_states, gate_w, gate_proj_w, up_proj_w, down_proj_w):
    B, S, H = hidden_states.shape
    x = hidden_states.reshape(-1, H)
    w = _router(x, gate_w)
    xb = x.astype(jnp.bfloat16)
    gpt = jnp.swapaxes(gate_proj_w, 1, 2).astype(jnp.bfloat16)
    upt = jnp.swapaxes(up_proj_w, 1, 2).astype(jnp.bfloat16)
    dpt = jnp.swapaxes(down_proj_w, 1, 2).astype(jnp.bfloat16)
    out = _ffn(xb, w, gpt, upt, dpt)
    return out.reshape(B, S, H)


# speedup vs baseline: 1.1418x; 1.1162x over previous
import functools

import jax
import jax.numpy as jnp
from jax.experimental import pallas as pl
from jax.experimental.pallas import tpu as pltpu

_T = 2048
_H = 2048
_DFF = 768
_E = 8
_EPAD = 128
_BT = 256
_BTF = 1024


def _router_body(x_ref, gwt_ref, w_ref):
    x = x_ref[...]
    logits = jax.lax.dot_general(
        x, gwt_ref[...], (((1,), (0,)), ((), ())),
        preferred_element_type=jnp.float32)
    col = jax.lax.broadcasted_iota(jnp.int32, (_BT, _EPAD), 1)
    valid = col < _E
    neg = jnp.float32(-1e30)
    logits = jnp.where(valid, logits, neg)
    m = jnp.max(logits, axis=1, keepdims=True)
    p = jnp.exp(logits - m)
    p = p / jnp.sum(p, axis=1, keepdims=True)
    p = jnp.where(valid, p, -1.0)
    m1 = jnp.max(p, axis=1, keepdims=True)
    i1 = jnp.min(jnp.where(p == m1, col, _EPAD), axis=1, keepdims=True)
    sel1 = col == i1
    p2 = jnp.where(sel1, -1.0, p)
    m2 = jnp.max(p2, axis=1, keepdims=True)
    i2 = jnp.min(jnp.where(p2 == m2, col, _EPAD), axis=1, keepdims=True)
    sel2 = col == i2
    denom = m1 + m2
    w1 = m1 / denom
    w2 = m2 / denom
    w_ref[...] = jnp.where(sel1, w1, 0.0) + jnp.where(sel2, w2, 0.0)


def _router(x, gate_w):
    gwt = jnp.zeros((_H, _EPAD), jnp.float32).at[:, :_E].set(gate_w.T)
    return pl.pallas_call(
        _router_body,
        grid=(_T // _BT,),
        in_specs=[
            pl.BlockSpec((_BT, _H), lambda i: (i, 0)),
            pl.BlockSpec((_H, _EPAD), lambda i: (0, 0)),
        ],
        out_specs=pl.BlockSpec((_BT, _EPAD), lambda i: (i, 0)),
        out_shape=jax.ShapeDtypeStruct((_T, _EPAD), jnp.float32),
    )(x, gwt)


def _ffn_body(x_ref, w_ref, gp_ref, up_ref, dp_ref, out_ref):
    e = pl.program_id(1)
    xb = x_ref[...]
    g = jax.lax.dot_general(
        xb, gp_ref[0], (((1,), (0,)), ((), ())),
        preferred_element_type=jnp.float32)
    u = jax.lax.dot_general(
        xb, up_ref[0], (((1,), (0,)), ((), ())),
        preferred_element_type=jnp.float32)
    h = (g * jax.lax.logistic(g) * u).astype(jnp.bfloat16)
    y = jax.lax.dot_general(
        h, dp_ref[0], (((1,), (0,)), ((), ())),
        preferred_element_type=jnp.float32)
    col = jax.lax.broadcasted_iota(jnp.int32, (_BTF, _EPAD), 1)
    we = jnp.sum(jnp.where(col == e, w_ref[...], 0.0), axis=1, keepdims=True)
    y = y * we

    @pl.when(e == 0)
    def _():
        out_ref[...] = y

    @pl.when(e != 0)
    def _():
        out_ref[...] += y


def _ffn(xb, w, gpt, upt, dpt):
    return pl.pallas_call(
        _ffn_body,
        grid=(_T // _BTF, _E),
        in_specs=[
            pl.BlockSpec((_BTF, _H), lambda i, e: (i, 0)),
            pl.BlockSpec((_BTF, _EPAD), lambda i, e: (i, 0)),
            pl.BlockSpec((1, _H, _DFF), lambda i, e: (e, 0, 0)),
            pl.BlockSpec((1, _H, _DFF), lambda i, e: (e, 0, 0)),
            pl.BlockSpec((1, _DFF, _H), lambda i, e: (e, 0, 0)),
        ],
        out_specs=pl.BlockSpec((_BTF, _H), lambda i, e: (i, 0)),
        out_shape=jax.ShapeDtypeStruct((_T, _H), jnp.float32),
        compiler_params=pltpu.CompilerParams(
            dimension_semantics=("arbitrary", "arbitrary")),
    )(xb, w, gpt, upt, dpt)


@jax.jit
def kernel(hidden_states, gate_w, gate_proj_w, up_proj_w, down_proj_w):
    B, S, H = hidden_states.shape
    x = hidden_states.reshape(-1, H)
    w = _router(x, gate_w)
    xb = x.astype(jnp.bfloat16)
    gpt = jnp.swapaxes(gate_proj_w, 1, 2).astype(jnp.bfloat16)
    upt = jnp.swapaxes(up_proj_w, 1, 2).astype(jnp.bfloat16)
    dpt = jnp.swapaxes(down_proj_w, 1, 2).astype(jnp.bfloat16)
    out = _ffn(xb, w, gpt, upt, dpt)
    return out.reshape(B, S, H)
